# baseline (speedup 1.0000x reference)
import functools

import jax
import jax.numpy as jnp
from jax import lax
from jax.experimental import pallas as pl
from jax.experimental.pallas import tpu as pltpu
from jax.experimental.pallas import tpu_sc as plsc

L = 16
K = 64
EB = 2000
DW = 16


def _sc_segment_sum(x, src, dst, *, n, d, e, nc, ns):
    half = n // nc
    ec = e // ns
    stripe = ((half + ns - 1) // ns + K - 1) // K * K
    accn = stripe * ns
    nzc = stripe // K
    last_rows = half - (ns - 1) * stripe

    mesh = plsc.VectorSubcoreMesh(
        core_axis_name="c", subcore_axis_name="s", num_cores=nc, num_subcores=ns
    )

    @functools.partial(
        pl.kernel,
        out_type=[
            jax.ShapeDtypeStruct((n, d), jnp.float32),
            jax.ShapeDtypeStruct((n, DW), jnp.float32),
        ],
        mesh=mesh,
        compiler_params=pltpu.CompilerParams(
            needs_layout_passes=False, use_tc_tiling_on_sc=False
        ),
        scratch_types=[
            pltpu.VMEM((EB,), jnp.int32),
            pltpu.VMEM((EB,), jnp.int32),
            pltpu.VMEM((EB + K,), jnp.int32),
            pltpu.VMEM((EB + K,), jnp.int32),
            pltpu.VMEM((K, d), jnp.float32),
            pltpu.VMEM((K, d), jnp.float32),
            pltpu.VMEM((K,), jnp.int32),
            pltpu.VMEM((K,), jnp.int32),
            pltpu.VMEM((K,), jnp.int32),
            pltpu.VMEM((K,), jnp.int32),
            pltpu.VMEM((K, DW), jnp.float32),
            pltpu.VMEM_SHARED((accn, d), jnp.float32),
            pltpu.VMEM_SHARED((accn, DW), jnp.float32),
            pltpu.SemaphoreType.DMA,
            pltpu.SemaphoreType.DMA,
        ],
    )
    def seg_kernel(x_h, src_h, dst_h, out_h, deg_h,
                   dst_b, src_b, kept_src, kept_dst, rows0, rows1,
                   sidx0, sidx1, gidx0, gidx1, ones_v, acc, dacc, sem0, sem1):
        c = lax.axis_index("c")
        s = lax.axis_index("s")
        lo = c * half

        zf = jnp.zeros((L,), jnp.float32)

        def zrow(r, _):
            def zcol(j, __):
                rows0[r, pl.ds(j * L, L)] = zf
                return 0
            lax.fori_loop(0, d // L, zcol, 0)
            ones_v[r, pl.ds(0, L)] = zf
            return 0
        lax.fori_loop(0, K, zrow, 0)
        for q in range(nzc):
            pltpu.sync_copy(rows0, acc.at[pl.ds(s * stripe + q * K, K)])
            pltpu.sync_copy(ones_v, dacc.at[pl.ds(s * stripe + q * K, K)])
        of = jnp.ones((L,), jnp.float32)

        def orow(r, _):
            ones_v[r, pl.ds(0, L)] = of
            return 0
        lax.fori_loop(0, K, orow, 0)

        plsc.subcore_barrier()

        def start_gather(gbuf, rbuf, sem_):
            pltpu.async_copy(x_h.at[gbuf], rbuf, sem_)

        def wait_gather(gbuf, rbuf, sem_):
            pltpu.make_async_copy(x_h.at[gbuf], rbuf, sem_).wait()

        def scatter(rbuf, sbuf):
            pltpu.sync_copy(rbuf, acc.at[sbuf], add=True)
            pltpu.sync_copy(ones_v, dacc.at[sbuf], add=True)

        def stage(sbuf, gbuf, cidx):
            base = cidx * K
            for j2 in range(K // L):
                sbuf[pl.ds(j2 * L, L)] = kept_dst[pl.ds(base + j2 * L, L)]
                gbuf[pl.ds(j2 * L, L)] = kept_src[pl.ds(base + j2 * L, L)]

        def fblock(b, _):
            pltpu.sync_copy(dst_h.at[pl.ds(s * ec + b * EB, EB)], dst_b)
            pltpu.sync_copy(src_h.at[pl.ds(s * ec + b * EB, EB)], src_b)

            def fbody(i, cnt):
                dv = dst_b[pl.ds(i * L, L)]
                sr = src_b[pl.ds(i * L, L)]
                m = (dv >= lo) & (dv < lo + half)
                mi = m.astype(jnp.int32)
                pos = cnt + plsc.cumsum(mi) - 1
                plsc.store_scatter(kept_src, [pos], sr, mask=m)
                plsc.store_scatter(kept_dst, [pos], dv - lo, mask=m)
                return cnt + jnp.sum(mi)
            cnt = lax.fori_loop(0, EB // L, fbody, jnp.int32(0))

            zi = jnp.zeros((L,), jnp.int32)
            dm = jnp.full((L,), accn - 1, jnp.int32)
            for j in range(K // L):
                kept_src[pl.ds(cnt + j * L, L)] = zi
                kept_dst[pl.ds(cnt + j * L, L)] = dm
            nch = (cnt + (K - 1)) // K

            @pl.when(nch > 0)
            def _():
                stage(sidx0, gidx0, 0)
                start_gather(gidx0, rows0, sem0)

            def pbody(p, __):
                a = 2 * p

                @pl.when(a + 1 < nch)
                def _():
                    stage(sidx1, gidx1, a + 1)
                    start_gather(gidx1, rows1, sem1)
                wait_gather(gidx0, rows0, sem0)
                scatter(rows0, sidx0)

                @pl.when(a + 2 < nch)
                def _():
                    stage(sidx0, gidx0, a + 2)
                    start_gather(gidx0, rows0, sem0)

                @pl.when(a + 1 < nch)
                def _():
                    wait_gather(gidx1, rows1, sem1)
                    scatter(rows1, sidx1)
                return 0
            lax.fori_loop(0, (nch + 1) // 2, pbody, 0)
            return 0
        lax.fori_loop(0, ec // EB, fblock, 0)

        plsc.subcore_barrier()

        @pl.when(s < ns - 1)
        def _():
            pltpu.sync_copy(acc.at[pl.ds(s * stripe, stripe)],
                            out_h.at[pl.ds(lo + s * stripe, stripe)])
            pltpu.sync_copy(dacc.at[pl.ds(s * stripe, stripe)],
                            deg_h.at[pl.ds(lo + s * stripe, stripe)])

        @pl.when(s == ns - 1)
        def _():
            pltpu.sync_copy(acc.at[pl.ds((ns - 1) * stripe, last_rows)],
                            out_h.at[pl.ds(lo + (ns - 1) * stripe, last_rows)])
            pltpu.sync_copy(dacc.at[pl.ds((ns - 1) * stripe, last_rows)],
                            deg_h.at[pl.ds(lo + (ns - 1) * stripe, last_rows)])

    return seg_kernel(x, src, dst)


def _tc_combine(x, summed, degv, wsT, wnT, bias, *, n, d, out):
    r_blk = 2000
    grid = n // r_blk

    def body(x_ref, a_ref, d_ref, ws_ref, wn_ref, b_ref, o_ref):
        deg = d_ref[:, :1]
        h = a_ref[...] * (1.0 / jnp.maximum(deg, 1.0))
        o_ref[...] = (
            jnp.dot(x_ref[...], ws_ref[...], preferred_element_type=jnp.float32)
            + jnp.dot(h, wn_ref[...], preferred_element_type=jnp.float32)
            + b_ref[...]
        )

    return pl.pallas_call(
        body,
        grid=(grid,),
        in_specs=[
            pl.BlockSpec((r_blk, d), lambda i: (i, 0)),
            pl.BlockSpec((r_blk, d), lambda i: (i, 0)),
            pl.BlockSpec((r_blk, DW), lambda i: (i, 0)),
            pl.BlockSpec((d, out), lambda i: (0, 0)),
            pl.BlockSpec((d, out), lambda i: (0, 0)),
            pl.BlockSpec((1, out), lambda i: (0, 0)),
        ],
        out_specs=pl.BlockSpec((r_blk, out), lambda i: (i, 0)),
        out_shape=jax.ShapeDtypeStruct((n, out), jnp.float32),
    )(x, summed, degv, wsT, wnT, bias)


def kernel(x, edge_index, W_self, b_self, W_neigh, b_neigh):
    n, d = x.shape
    e = edge_index.shape[1]
    out = W_self.shape[0]

    src = edge_index[0]
    dst = edge_index[1]

    summed, degv = _sc_segment_sum(x, src, dst, n=n, d=d, e=e, nc=2, ns=16)

    bias = (b_self + b_neigh)[None, :]
    return _tc_combine(x, summed, degv, W_self.T, W_neigh.T, bias,
                       n=n, d=d, out=out)
